# baseline (speedup 1.0000x reference)
import functools

import jax
import jax.numpy as jnp
from jax import lax
from jax.experimental import pallas as pl
from jax.experimental.pallas import tpu as pltpu
from jax.experimental.pallas import tpu_sc as plsc

_BSZ = 32


def _broadcast_kernel(num_prefix: int, hidden: int):
    info = plsc.get_sparse_core_info()
    num_cores, num_subcores = info.num_cores, info.num_subcores
    num_workers = num_cores * num_subcores
    assert _BSZ % num_workers == 0 or num_workers % _BSZ == 0
    rows_per_sub = num_prefix // num_subcores
    mesh = plsc.VectorSubcoreMesh(core_axis_name="c", subcore_axis_name="s")

    @functools.partial(
        pl.kernel,
        mesh=mesh,
        out_type=jax.ShapeDtypeStruct((_BSZ, num_prefix, hidden), jnp.float32),
    )
    def body(table_hbm, out_hbm):
        cid = lax.axis_index("c")
        sid = lax.axis_index("s")
        wid = cid * num_subcores + sid
        pltpu.sync_copy(table_hbm, out_hbm.at[wid])

    return body


def kernel(bsz, prefix_weight):
    num_prefix, hidden = prefix_weight.shape
    return _broadcast_kernel(num_prefix, hidden)(prefix_weight)


# speedup vs baseline: 49.5033x; 49.5033x over previous
import functools

import jax
import jax.numpy as jnp
from jax import lax
from jax.experimental import pallas as pl
from jax.experimental.pallas import tpu as pltpu
from jax.experimental.pallas import tpu_sc as plsc

_BSZ = 32


def _broadcast_kernel(num_prefix: int, hidden: int):
    info = plsc.get_sparse_core_info()
    num_cores, num_subcores = info.num_cores, info.num_subcores
    num_workers = num_cores * num_subcores
    assert num_prefix % num_workers == 0
    rows_per_w = num_prefix // num_workers
    mesh = plsc.VectorSubcoreMesh(core_axis_name="c", subcore_axis_name="s")

    @functools.partial(
        pl.kernel,
        mesh=mesh,
        out_type=jax.ShapeDtypeStruct((_BSZ, num_prefix, hidden), jnp.float32),
        scratch_types=[
            pltpu.VMEM((rows_per_w, hidden), jnp.float32),
            pltpu.SemaphoreType.DMA,
        ],
    )
    def body(table_hbm, out_hbm, stripe_v, sem):
        cid = lax.axis_index("c")
        sid = lax.axis_index("s")
        wid = cid * num_subcores + sid
        row0 = wid * rows_per_w
        pltpu.sync_copy(table_hbm.at[pl.ds(row0, rows_per_w)], stripe_v)
        copies = [
            pltpu.async_copy(
                stripe_v, out_hbm.at[b, pl.ds(row0, rows_per_w)], sem
            )
            for b in range(_BSZ)
        ]
        for c in copies:
            c.wait()

    return body


def kernel(bsz, prefix_weight):
    num_prefix, hidden = prefix_weight.shape
    return _broadcast_kernel(num_prefix, hidden)(prefix_weight)
